# baseline (speedup 1.0000x reference)
import jax
import jax.numpy as jnp
from jax import lax
from jax.experimental import pallas as pl
from jax.experimental.pallas import tpu as pltpu
from jax.experimental.pallas import tpu_sc as plsc

NC = 2
NS = 16
NW = NC * NS

D = 64
IDX_MINOR = 128
K = 4
CHUNK = K * IDX_MINOR


def _emb_body(idx_hbm, table_hbm, out_hbm, idx_v, rows_v, gsem):
  wid = lax.axis_index("s") * NC + lax.axis_index("c")
  nrows = idx_hbm.shape[0]
  rows_per_w = nrows // NW
  chunks = rows_per_w // K
  row0 = wid * rows_per_w

  def chunk_body(i, carry):
    r = row0 + i * K
    pltpu.sync_copy(idx_hbm.at[pl.ds(r, K)], idx_v)
    cps = [
        pltpu.async_copy(table_hbm.at[idx_v.at[j]],
                         rows_v.at[pl.ds(j * IDX_MINOR, IDX_MINOR)], gsem)
        for j in range(K)
    ]
    for cp in cps:
      cp.wait()
    pltpu.sync_copy(rows_v, out_hbm.at[pl.ds(r * IDX_MINOR, CHUNK)])
    return carry

  lax.fori_loop(0, chunks, chunk_body, 0)


def kernel(src, seg, table):
  del seg
  B, S = src.shape
  n = B * S
  idx = src.reshape(n // IDX_MINOR, IDX_MINOR).astype(jnp.int32)
  mesh = plsc.VectorSubcoreMesh(core_axis_name="c", subcore_axis_name="s",
                                num_cores=NC, num_subcores=NS)
  run = pl.kernel(
      _emb_body,
      out_type=jax.ShapeDtypeStruct((n, D), jnp.float32),
      mesh=mesh,
      scratch_types=[
          pltpu.VMEM((K, IDX_MINOR), jnp.int32),
          pltpu.VMEM((CHUNK, D), jnp.float32),
          pltpu.SemaphoreType.DMA,
      ],
      compiler_params=pltpu.CompilerParams(use_tc_tiling_on_sc=False),
  )
  out = run(idx, table)
  return out.reshape(B, S, D)


# speedup vs baseline: 1.0296x; 1.0296x over previous
import jax
import jax.numpy as jnp
from jax import lax
from jax.experimental import pallas as pl
from jax.experimental.pallas import tpu as pltpu
from jax.experimental.pallas import tpu_sc as plsc

NC = 2
NS = 16
NW = NC * NS

D = 64
IDX_MINOR = 128
K = 4
CHUNK = K * IDX_MINOR
NBUF = 2


def _emb_body(idx_hbm, table_hbm, out_hbm, idx_v, rows_v,
              gsem0, gsem1, wsem0, wsem1):
  gsems = (gsem0, gsem1)
  wsems = (wsem0, wsem1)
  wid = lax.axis_index("s") * NC + lax.axis_index("c")
  nrows = idx_hbm.shape[0]
  rows_per_w = nrows // NW
  chunks = rows_per_w // K
  row0 = wid * rows_per_w

  def load_idx(i, b):
    pltpu.sync_copy(idx_hbm.at[pl.ds(row0 + i * K, K)], idx_v.at[b])

  def fire_gather(b):
    for j in range(K):
      pltpu.async_copy(table_hbm.at[idx_v.at[b].at[j]],
                       rows_v.at[b].at[pl.ds(j * IDX_MINOR, IDX_MINOR)],
                       gsems[b])

  def wait_gather(b):
    for j in range(K):
      pltpu.make_async_copy(table_hbm.at[idx_v.at[b].at[j]],
                            rows_v.at[b].at[pl.ds(j * IDX_MINOR, IDX_MINOR)],
                            gsems[b]).wait()

  def fire_wb(i, b):
    pltpu.async_copy(rows_v.at[b],
                     out_hbm.at[pl.ds((row0 + i * K) * IDX_MINOR, CHUNK)],
                     wsems[b])

  def wait_wb(i, b):
    pltpu.make_async_copy(rows_v.at[b],
                          out_hbm.at[pl.ds((row0 + i * K) * IDX_MINOR, CHUNK)],
                          wsems[b]).wait()

  load_idx(0, 0)
  fire_gather(0)

  wait_gather(0)
  fire_wb(0, 0)
  load_idx(1, 1)
  fire_gather(1)

  wait_gather(1)
  fire_wb(1, 1)
  load_idx(2, 0)
  wait_wb(0, 0)
  fire_gather(0)

  def pair_body(p, carry):
    for db in range(NBUF):
      i = 2 * p + db
      b = db
      nb = 1 - db
      wait_gather(b)
      fire_wb(i, b)
      load_idx(i + 1, nb)
      wait_wb(i - 1, nb)
      fire_gather(nb)
    return carry

  lax.fori_loop(1, chunks // 2 - 1, pair_body, 0)

  wait_gather(0)
  fire_wb(chunks - 2, 0)
  load_idx(chunks - 1, 1)
  wait_wb(chunks - 3, 1)
  fire_gather(1)

  wait_gather(1)
  fire_wb(chunks - 1, 1)
  wait_wb(chunks - 2, 0)
  wait_wb(chunks - 1, 1)


def kernel(src, seg, table):
  del seg
  B, S = src.shape
  n = B * S
  idx = src.reshape(n // IDX_MINOR, IDX_MINOR).astype(jnp.int32)
  mesh = plsc.VectorSubcoreMesh(core_axis_name="c", subcore_axis_name="s",
                                num_cores=NC, num_subcores=NS)
  run = pl.kernel(
      _emb_body,
      out_type=jax.ShapeDtypeStruct((n, D), jnp.float32),
      mesh=mesh,
      scratch_types=[
          pltpu.VMEM((NBUF, K, IDX_MINOR), jnp.int32),
          pltpu.VMEM((NBUF, CHUNK, D), jnp.float32),
          pltpu.SemaphoreType.DMA,
          pltpu.SemaphoreType.DMA,
          pltpu.SemaphoreType.DMA,
          pltpu.SemaphoreType.DMA,
      ],
      compiler_params=pltpu.CompilerParams(use_tc_tiling_on_sc=False),
  )
  out = run(idx, table)
  return out.reshape(B, S, D)
